# baseline (speedup 1.0000x reference)
import functools

import jax
import jax.numpy as jnp
from jax import lax
from jax.experimental import pallas as pl
from jax.experimental.pallas import tpu as pltpu
from jax.experimental.pallas import tpu_sc as plsc

D = 768
PERIOD = 86400

NW = 32
SCH = 4
NB = 4
ROWS = NB * SCH
NCH = 64
S_PER_W = NCH * SCH


def _fused_body(ts_hbm, x_hbm, pe_hbm, table_hbm, out_hbm,
                idx_v, xb0, xb1, xb2, xb3, cb0, cb1, pb0, pb1,
                si0, si1, sw0, sw1, sw2, sw3):
    wid = lax.axis_index("s") * 2 + lax.axis_index("c")
    s_base = wid * S_PER_W
    xb = (xb0, xb1, xb2, xb3)
    cb = (cb0, cb1)
    pb = (pb0, pb1)
    semi = (si0, si1)
    semw = (sw0, sw1, sw2, sw3)

    pltpu.sync_copy(ts_hbm.at[wid], idx_v)
    @pl.loop(0, (NCH * ROWS) // 16)
    def _mod_loop(i):
        sl = pl.ds(i * 16, 16)
        t = idx_v[sl]
        r = lax.rem(t, PERIOD)
        idx_v[sl] = jnp.minimum(jnp.maximum(r, 0), PERIOD - 1)

    def start_in(cc, slot, par):
        s0 = s_base + cc * SCH
        for b in range(NB):
            pltpu.async_copy(x_hbm.at[b, pl.ds(s0, SCH)], xb[slot].at[b],
                             semi[par])
        isl = idx_v.at[pl.ds(cc * ROWS, ROWS)]
        pltpu.async_copy(table_hbm.at[isl], cb[par], semi[par])
        pltpu.async_copy(pe_hbm.at[pl.ds(s0, SCH)], pb[par], semi[par])

    def drain_in(slot, par):
        for b in range(NB):
            pltpu.make_async_copy(x_hbm.at[b, pl.ds(0, SCH)],
                                  xb[slot].at[b], semi[par]).wait()
        pltpu.make_async_copy(table_hbm.at[pl.ds(0, ROWS)], cb[par],
                              semi[par]).wait()
        pltpu.make_async_copy(pe_hbm.at[pl.ds(0, SCH)], pb[par],
                              semi[par]).wait()

    def compute(slot, par):
        for j in range(SCH):
            @pl.loop(0, D // 16)
            def _v_loop(v):
                dsv = pl.ds(v * 16, 16)
                pv = pb[par][j, dsv]
                for b in range(NB):
                    xb[slot][b, j, dsv] = (xb[slot][b, j, dsv]
                                           + cb[par][b * SCH + j, dsv] + pv)

    def start_wb(cc, slot):
        s0 = s_base + cc * SCH
        for b in range(NB):
            pltpu.async_copy(xb[slot].at[b], out_hbm.at[b, pl.ds(s0, SCH)],
                             semw[slot])

    def drain_wb(slot):
        for b in range(NB):
            pltpu.make_async_copy(xb[slot].at[b],
                                  out_hbm.at[b, pl.ds(0, SCH)],
                                  semw[slot]).wait()

    def phase(cc, ph, do_wb_drain, do_in):
        slot, par = ph, ph & 1
        drain_in(slot, par)
        compute(slot, par)
        start_wb(cc, slot)
        if do_in:
            if do_wb_drain:
                drain_wb((ph + 2) % 4)
            start_in(cc + 2, (ph + 2) % 4, par)

    start_in(0, 0, 0)
    start_in(1, 1, 1)
    phase(0, 0, False, True)
    phase(1, 1, False, True)
    phase(2, 2, True, True)
    phase(3, 3, True, True)

    @pl.loop(4, NCH - 4, step=4)
    def _main(c):
        for ph in range(4):
            phase(c + ph, ph, True, True)

    phase(NCH - 4, 0, True, True)
    phase(NCH - 3, 1, True, True)
    phase(NCH - 2, 2, False, False)
    phase(NCH - 1, 3, False, False)
    for slot in range(4):
        drain_wb(slot)


def _fused(ts, x, pe, table):
    B, S, _ = x.shape
    k = pl.kernel(
        _fused_body,
        out_type=jax.ShapeDtypeStruct((B, S, D), jnp.float32),
        mesh=plsc.VectorSubcoreMesh(core_axis_name="c", subcore_axis_name="s"),
        scratch_types=[
            pltpu.VMEM((NCH * ROWS,), jnp.int32),
            pltpu.VMEM((NB, SCH, D), jnp.float32),
            pltpu.VMEM((NB, SCH, D), jnp.float32),
            pltpu.VMEM((NB, SCH, D), jnp.float32),
            pltpu.VMEM((NB, SCH, D), jnp.float32),
            pltpu.VMEM((ROWS, D), jnp.float32),
            pltpu.VMEM((ROWS, D), jnp.float32),
            pltpu.VMEM((SCH, D), jnp.float32),
            pltpu.VMEM((SCH, D), jnp.float32),
            pltpu.SemaphoreType.DMA,
            pltpu.SemaphoreType.DMA,
            pltpu.SemaphoreType.DMA,
            pltpu.SemaphoreType.DMA,
            pltpu.SemaphoreType.DMA,
            pltpu.SemaphoreType.DMA,
        ],
    )
    return k(ts, x, pe, table)


def kernel(x, timestamps, pe, circadian_pe):
    B, S, d = x.shape
    assert d == D and B == NB and S == NW * S_PER_W
    ts = timestamps.astype(jnp.int32)
    ts_perm = (ts.reshape(B, NW, NCH, SCH)
               .transpose(1, 2, 0, 3)
               .reshape(NW, NCH * ROWS))
    return _fused(ts_perm, x, pe[:S], circadian_pe)
